# baseline (speedup 1.0000x reference)
import jax
import jax.numpy as jnp
from jax.experimental import pallas as pl
from jax.experimental.pallas import tpu as pltpu

_T = 512
_C = 512
_H = 8
_D = 64
_D2 = _D // 2
_R = 32


def _attn_pair_kernel(x_ref, cos_ref, sin_ref, wq_ref, wk_ref, wv_ref, y_ref,
                      q_s, kt_s, v_s):
    x = x_ref[...]
    c = cos_ref[...]
    s = sin_ref[...]

    def rot_norm(p):
        p1 = p[:, :_D2]
        p2 = p[:, _D2:]
        r1 = p1 * c + p2 * s
        r2 = p2 * c - p1 * s
        r = jnp.concatenate([r1, r2], axis=-1)
        ms = jnp.mean(r * r, axis=-1, keepdims=True)
        return r * jax.lax.rsqrt(ms + 1e-6)

    for sub in range(2):
        lo, hi = sub * _D, (sub + 1) * _D
        q = rot_norm(jnp.dot(x, wq_ref[:, lo:hi], preferred_element_type=jnp.float32))
        k = rot_norm(jnp.dot(x, wk_ref[:, lo:hi], preferred_element_type=jnp.float32))
        q_s[sub] = q
        kt_s[sub] = k.T
        v_s[sub] = jnp.dot(x, wv_ref[:, lo:hi], preferred_element_type=jnp.float32)

    for sub in range(2):
        lo, hi = sub * _D, (sub + 1) * _D
        for ib in range(_T // _R):
            jmax = min(_T, (((ib + 1) * _R + 127) // 128) * 128)
            qc = q_s[sub, ib * _R : (ib + 1) * _R, :]
            sc = qc[:, 0:1] + kt_s[sub, 0:1, :jmax]
            for d in range(1, _D):
                sc = jnp.maximum(sc, qc[:, d : d + 1] + kt_s[sub, d : d + 1, :jmax])
            row = jax.lax.broadcasted_iota(jnp.int32, (_R, jmax), 0) + ib * _R
            col = jax.lax.broadcasted_iota(jnp.int32, (_R, jmax), 1)
            sc = jnp.where(row >= col, sc, jnp.float32(-1e30))
            m = jnp.max(sc, axis=-1, keepdims=True)
            p = jnp.exp(sc - m)
            denom = jnp.sum(p, axis=-1, keepdims=True)
            w = p / denom
            y_ref[ib * _R : (ib + 1) * _R, lo:hi] = jnp.dot(
                w, v_s[sub, :jmax, :], preferred_element_type=jnp.float32
            )


def _out_proj_kernel(y_ref, wo_ref, o_ref):
    yv = y_ref[...]
    acc = jnp.dot(yv[:, 0 : 2 * _D], wo_ref[0], preferred_element_type=jnp.float32)
    for g in range(1, 4):
        acc += jnp.dot(
            yv[:, g * 2 * _D : (g + 1) * 2 * _D],
            wo_ref[g],
            preferred_element_type=jnp.float32,
        )
    o_ref[...] = acc


def kernel(x, cos, sin, Wq, Wk, Wv, Wo):
    B = x.shape[0]
    x2 = x.reshape(_T, _C)
    wo4 = Wo.reshape(4, 2 * _D, _C)

    y = pl.pallas_call(
        _attn_pair_kernel,
        grid=(4,),
        in_specs=[
            pl.BlockSpec((_T, _C), lambda p: (0, 0)),
            pl.BlockSpec((_T, _D2), lambda p: (0, 0)),
            pl.BlockSpec((_T, _D2), lambda p: (0, 0)),
            pl.BlockSpec((_C, 2 * _D), lambda p: (0, p)),
            pl.BlockSpec((_C, 2 * _D), lambda p: (0, p)),
            pl.BlockSpec((_C, 2 * _D), lambda p: (0, p)),
        ],
        out_specs=pl.BlockSpec((_T, 2 * _D), lambda p: (0, p)),
        out_shape=jax.ShapeDtypeStruct((_T, _C), jnp.float32),
        scratch_shapes=[
            pltpu.VMEM((2, _T, _D), jnp.float32),
            pltpu.VMEM((2, _D, _T), jnp.float32),
            pltpu.VMEM((2, _T, _D), jnp.float32),
        ],
        compiler_params=pltpu.CompilerParams(
            dimension_semantics=("parallel",),
            vmem_limit_bytes=56 * 1024 * 1024,
        ),
    )(x2, cos, sin, Wq, Wk, Wv)

    out = pl.pallas_call(
        _out_proj_kernel,
        grid=(2,),
        in_specs=[
            pl.BlockSpec((_T // 2, _C), lambda i: (i, 0), memory_space=pltpu.VMEM),
            pl.BlockSpec((4, 2 * _D, _C), lambda i: (0, 0, 0)),
        ],
        out_specs=pl.BlockSpec((_T // 2, _C), lambda i: (i, 0)),
        out_shape=jax.ShapeDtypeStruct((_T, _C), jnp.float32),
        compiler_params=pltpu.CompilerParams(
            dimension_semantics=("parallel",),
        ),
    )(y, wo4)
    return out.reshape(B, _T, _C)


# speedup vs baseline: 1.2527x; 1.2527x over previous
import jax
import jax.numpy as jnp
from jax.experimental import pallas as pl
from jax.experimental.pallas import tpu as pltpu

_T = 512
_C = 512
_H = 8
_D = 64
_D2 = _D // 2
_R = 64


def _attn_pair_kernel(x_ref, cos_ref, sin_ref, wq_ref, wk_ref, wv_ref, y_ref,
                      q_s, kt_s, v_s):
    x = x_ref[...]
    c = cos_ref[...]
    s = sin_ref[...]

    def rot_norm(p):
        p1 = p[:, :_D2]
        p2 = p[:, _D2:]
        r1 = p1 * c + p2 * s
        r2 = p2 * c - p1 * s
        r = jnp.concatenate([r1, r2], axis=-1)
        ms = jnp.mean(r * r, axis=-1, keepdims=True)
        return r * jax.lax.rsqrt(ms + 1e-6)

    for sub in range(2):
        lo, hi = sub * _D, (sub + 1) * _D
        q = rot_norm(jnp.dot(x, wq_ref[:, lo:hi], preferred_element_type=jnp.float32))
        k = rot_norm(jnp.dot(x, wk_ref[:, lo:hi], preferred_element_type=jnp.float32))
        q_s[sub] = q
        kt_s[sub] = k.T
        v_s[sub] = jnp.dot(x, wv_ref[:, lo:hi], preferred_element_type=jnp.float32)

    for sub in range(2):
        lo, hi = sub * _D, (sub + 1) * _D
        for ib in range(_T // _R):
            jmax = min(_T, (((ib + 1) * _R + 127) // 128) * 128)
            qc = q_s[sub, ib * _R : (ib + 1) * _R, :]
            sc = qc[:, 0:1] + kt_s[sub, 0:1, :jmax]
            for d in range(1, _D):
                sc = jnp.maximum(sc, qc[:, d : d + 1] + kt_s[sub, d : d + 1, :jmax])
            row = jax.lax.broadcasted_iota(jnp.int32, (_R, jmax), 0) + ib * _R
            col = jax.lax.broadcasted_iota(jnp.int32, (_R, jmax), 1)
            sc = jnp.where(row >= col, sc, jnp.float32(-1e30))
            m = jnp.max(sc, axis=-1, keepdims=True)
            p = jnp.exp(sc - m)
            denom = jnp.sum(p, axis=-1, keepdims=True)
            w = p / denom
            y_ref[ib * _R : (ib + 1) * _R, lo:hi] = jnp.dot(
                w, v_s[sub, :jmax, :], preferred_element_type=jnp.float32
            )


def _out_proj_kernel(y_ref, wo_ref, o_ref):
    yv = y_ref[...]
    acc = jnp.dot(yv[:, 0 : 2 * _D], wo_ref[0], preferred_element_type=jnp.float32)
    for g in range(1, 4):
        acc += jnp.dot(
            yv[:, g * 2 * _D : (g + 1) * 2 * _D],
            wo_ref[g],
            preferred_element_type=jnp.float32,
        )
    o_ref[...] = acc


def kernel(x, cos, sin, Wq, Wk, Wv, Wo):
    B = x.shape[0]
    x2 = x.reshape(_T, _C)
    wo4 = Wo.reshape(4, 2 * _D, _C)

    y = pl.pallas_call(
        _attn_pair_kernel,
        grid=(4,),
        in_specs=[
            pl.BlockSpec((_T, _C), lambda p: (0, 0)),
            pl.BlockSpec((_T, _D2), lambda p: (0, 0)),
            pl.BlockSpec((_T, _D2), lambda p: (0, 0)),
            pl.BlockSpec((_C, 2 * _D), lambda p: (0, p)),
            pl.BlockSpec((_C, 2 * _D), lambda p: (0, p)),
            pl.BlockSpec((_C, 2 * _D), lambda p: (0, p)),
        ],
        out_specs=pl.BlockSpec((_T, 2 * _D), lambda p: (0, p)),
        out_shape=jax.ShapeDtypeStruct((_T, _C), jnp.float32),
        scratch_shapes=[
            pltpu.VMEM((2, _T, _D), jnp.float32),
            pltpu.VMEM((2, _D, _T), jnp.float32),
            pltpu.VMEM((2, _T, _D), jnp.float32),
        ],
        compiler_params=pltpu.CompilerParams(
            dimension_semantics=("parallel",),
            vmem_limit_bytes=56 * 1024 * 1024,
        ),
    )(x2, cos, sin, Wq, Wk, Wv)

    out = pl.pallas_call(
        _out_proj_kernel,
        grid=(2,),
        in_specs=[
            pl.BlockSpec((_T // 2, _C), lambda i: (i, 0), memory_space=pltpu.VMEM),
            pl.BlockSpec((4, 2 * _D, _C), lambda i: (0, 0, 0)),
        ],
        out_specs=pl.BlockSpec((_T // 2, _C), lambda i: (i, 0)),
        out_shape=jax.ShapeDtypeStruct((_T, _C), jnp.float32),
        compiler_params=pltpu.CompilerParams(
            dimension_semantics=("parallel",),
        ),
    )(y, wo4)
    return out.reshape(B, _T, _C)


# speedup vs baseline: 1.3999x; 1.1174x over previous
import jax
import jax.numpy as jnp
from jax.experimental import pallas as pl
from jax.experimental.pallas import tpu as pltpu

_T = 512
_C = 512
_H = 8
_D = 64
_D2 = _D // 2
_R = 128


def _attn_pair_kernel(x_ref, cos_ref, sin_ref, wq_ref, wk_ref, wv_ref, y_ref,
                      q_s, kt_s, v_s, qb_s):
    x = x_ref[...]
    c = cos_ref[...]
    s = sin_ref[...]

    def rot_norm(p):
        p1 = p[:, :_D2]
        p2 = p[:, _D2:]
        r1 = p1 * c + p2 * s
        r2 = p2 * c - p1 * s
        r = jnp.concatenate([r1, r2], axis=-1)
        ms = jnp.mean(r * r, axis=-1, keepdims=True)
        return r * jax.lax.rsqrt(ms + 1e-6)

    for sub in range(2):
        lo, hi = sub * _D, (sub + 1) * _D
        q = rot_norm(jnp.dot(x, wq_ref[:, lo:hi], preferred_element_type=jnp.float32))
        k = rot_norm(jnp.dot(x, wk_ref[:, lo:hi], preferred_element_type=jnp.float32))
        q_s[sub] = q
        kt_s[sub] = k.T
        v_s[sub] = jnp.dot(x, wv_ref[:, lo:hi], preferred_element_type=jnp.float32)

    for sub in range(2):
        lo, hi = sub * _D, (sub + 1) * _D
        for ib in range(_T // _R):
            jmax = min(_T, (((ib + 1) * _R + 127) // 128) * 128)
            nj = jmax // 128
            qc = q_s[sub, ib * _R : (ib + 1) * _R, :]
            for d in range(_D):
                qb_s[d] = jnp.broadcast_to(qc[:, d : d + 1], (_R, 128))
            blocks = []
            for jb in range(nj):
                jl, jh = jb * 128, (jb + 1) * 128
                sc = qb_s[0] + kt_s[sub, 0:1, jl:jh]
                for d in range(1, _D):
                    sc = jnp.maximum(sc, qb_s[d] + kt_s[sub, d : d + 1, jl:jh])
                blocks.append(sc)
            sc = blocks[0] if nj == 1 else jnp.concatenate(blocks, axis=1)
            row = jax.lax.broadcasted_iota(jnp.int32, (_R, jmax), 0) + ib * _R
            col = jax.lax.broadcasted_iota(jnp.int32, (_R, jmax), 1)
            sc = jnp.where(row >= col, sc, jnp.float32(-1e30))
            m = jnp.max(sc, axis=-1, keepdims=True)
            p = jnp.exp(sc - m)
            denom = jnp.sum(p, axis=-1, keepdims=True)
            w = p / denom
            y_ref[ib * _R : (ib + 1) * _R, lo:hi] = jnp.dot(
                w, v_s[sub, :jmax, :], preferred_element_type=jnp.float32
            )


def _out_proj_kernel(y_ref, wo_ref, o_ref):
    yv = y_ref[...]
    acc = jnp.dot(yv[:, 0 : 2 * _D], wo_ref[0], preferred_element_type=jnp.float32)
    for g in range(1, 4):
        acc += jnp.dot(
            yv[:, g * 2 * _D : (g + 1) * 2 * _D],
            wo_ref[g],
            preferred_element_type=jnp.float32,
        )
    o_ref[...] = acc


def kernel(x, cos, sin, Wq, Wk, Wv, Wo):
    B = x.shape[0]
    x2 = x.reshape(_T, _C)
    wo4 = Wo.reshape(4, 2 * _D, _C)

    y = pl.pallas_call(
        _attn_pair_kernel,
        grid=(4,),
        in_specs=[
            pl.BlockSpec((_T, _C), lambda p: (0, 0)),
            pl.BlockSpec((_T, _D2), lambda p: (0, 0)),
            pl.BlockSpec((_T, _D2), lambda p: (0, 0)),
            pl.BlockSpec((_C, 2 * _D), lambda p: (0, p)),
            pl.BlockSpec((_C, 2 * _D), lambda p: (0, p)),
            pl.BlockSpec((_C, 2 * _D), lambda p: (0, p)),
        ],
        out_specs=pl.BlockSpec((_T, 2 * _D), lambda p: (0, p)),
        out_shape=jax.ShapeDtypeStruct((_T, _C), jnp.float32),
        scratch_shapes=[
            pltpu.VMEM((2, _T, _D), jnp.float32),
            pltpu.VMEM((2, _D, _T), jnp.float32),
            pltpu.VMEM((2, _T, _D), jnp.float32),
            pltpu.VMEM((_D, _R, 128), jnp.float32),
        ],
        compiler_params=pltpu.CompilerParams(
            dimension_semantics=("arbitrary",),
            vmem_limit_bytes=56 * 1024 * 1024,
        ),
    )(x2, cos, sin, Wq, Wk, Wv)

    out = pl.pallas_call(
        _out_proj_kernel,
        grid=(2,),
        in_specs=[
            pl.BlockSpec((_T // 2, _C), lambda i: (i, 0), memory_space=pltpu.VMEM),
            pl.BlockSpec((4, 2 * _D, _C), lambda i: (0, 0, 0)),
        ],
        out_specs=pl.BlockSpec((_T // 2, _C), lambda i: (i, 0)),
        out_shape=jax.ShapeDtypeStruct((_T, _C), jnp.float32),
        compiler_params=pltpu.CompilerParams(
            dimension_semantics=("arbitrary",),
        ),
    )(y, wo4)
    return out.reshape(B, _T, _C)
